# baseline (speedup 1.0000x reference)
import functools

import jax
import jax.numpy as jnp
from jax import lax
from jax.experimental import pallas as pl
from jax.experimental.pallas import tpu as pltpu
from jax.experimental.pallas import tpu_sc as plsc

B, L, D = 1024, 200, 64
N = B * L
NC, NS, LANES = 2, 16, 16
NW = NC * NS
TPW = N // NW
GS = 128
NG = TPW // GS
NSTR = GS // LANES


def _sc_embed(tok_pairs, tidx2, par2, aidx2, peseg):
    mesh = plsc.VectorSubcoreMesh(core_axis_name="c", subcore_axis_name="s")

    @functools.partial(
        pl.kernel,
        mesh=mesh,
        compiler_params=pltpu.CompilerParams(use_tc_tiling_on_sc=True,
                                             needs_layout_passes=False),
        out_type=jax.ShapeDtypeStruct((N // 2, 2 * D), jnp.float32),
        scratch_types=[
            pltpu.VMEM((TPW,), jnp.int32),
            pltpu.VMEM((TPW,), jnp.int32),
            pltpu.VMEM((TPW,), jnp.int32),
            pltpu.VMEM((GS, 2 * D), jnp.float32),
            pltpu.VMEM((GS, 2 * D), jnp.float32),
            pltpu.VMEM((GS // 2, 2 * D), jnp.float32),
            pltpu.SemaphoreType.DMA,
            pltpu.SemaphoreType.DMA,
        ],
    )
    def k(tok_hbm, tidx_hbm, par_hbm, aidx_hbm, peseg_hbm, out_hbm,
          tidx_v, par_v, aidx_v, tok_v, add_v, res_v, sem_t, sem_a):
        wid = lax.axis_index("s") * NC + lax.axis_index("c")
        pltpu.sync_copy(tidx_hbm.at[wid], tidx_v)
        pltpu.sync_copy(par_hbm.at[wid], par_v)
        pltpu.sync_copy(aidx_hbm.at[wid], aidx_v)

        def group(g, carry):
            gbase = g * GS
            cp_t = pltpu.async_copy(tok_hbm.at[tidx_v.at[pl.ds(gbase, GS)]],
                                    tok_v, sem_t)
            cp_a = pltpu.async_copy(peseg_hbm.at[aidx_v.at[pl.ds(gbase, GS)]],
                                    add_v, sem_a)
            cp_t.wait()
            cp_a.wait()

            def stripe(s, c2):
                r_vec = s * LANES + lax.iota(jnp.int32, LANES)
                par_vec = par_v[pl.ds(gbase + s * LANES, LANES)]
                tcol0 = par_vec * D
                drow = lax.shift_right_logical(r_vec, 1)
                dcol0 = (r_vec & 1) * D
                zero = r_vec * 0

                def col(j, c3):
                    tv = plsc.load_gather(tok_v, [r_vec, tcol0 + j])
                    av = plsc.load_gather(add_v, [r_vec, zero + j])
                    plsc.store_scatter(res_v, [drow, dcol0 + j], tv + av)
                    return c3

                lax.fori_loop(0, D, col, 0, unroll=4)
                return c2

            lax.fori_loop(0, NSTR, stripe, 0)
            off = pl.multiple_of(wid * (TPW // 2) + g * (GS // 2), 8)
            pltpu.sync_copy(res_v, out_hbm.at[pl.ds(off, GS // 2)])
            return carry

        lax.fori_loop(0, NG, group, 0)

    return k(tok_pairs, tidx2, par2, aidx2, peseg)


def kernel(sequence, segment_labels, tok_table, seg_table, pe):
    tok_pairs = tok_table.reshape(tok_table.shape[0] // 2, 2 * D)
    seq = sequence.astype(jnp.int32)
    tidx2 = (seq // 2).reshape(NW, TPW)
    par2 = (seq & 1).reshape(NW, TPW)
    l_pos = jnp.arange(L, dtype=jnp.int32)
    aidx2 = (segment_labels.astype(jnp.int32) * L + l_pos[None, :]).reshape(NW, TPW)
    peseg = (seg_table[:, None, :] + pe[0, :L, :][None, :, :]).reshape(3 * L, D)
    peseg = jnp.concatenate([peseg, peseg], axis=1)
    out = _sc_embed(tok_pairs, tidx2, par2, aidx2, peseg)
    return out.reshape(B, L, D)


# speedup vs baseline: 1.7098x; 1.7098x over previous
import functools

import jax
import jax.numpy as jnp
from jax import lax
from jax.experimental import pallas as pl
from jax.experimental.pallas import tpu as pltpu
from jax.experimental.pallas import tpu_sc as plsc

B, L, D = 1024, 200, 64
N = B * L
NC, NS, LANES = 2, 16, 16
NW = NC * NS
TPW = N // NW
GS = 128
NG = TPW // GS


def _sc_embed(tok128, tidx2, aidx2, peseg):
    mesh = plsc.VectorSubcoreMesh(core_axis_name="c", subcore_axis_name="s")

    @functools.partial(
        pl.kernel,
        mesh=mesh,
        compiler_params=pltpu.CompilerParams(use_tc_tiling_on_sc=True,
                                             needs_layout_passes=False),
        out_type=jax.ShapeDtypeStruct((N // 2, 2 * D), jnp.float32),
        scratch_types=[
            pltpu.VMEM((TPW,), jnp.int32),
            pltpu.VMEM((TPW,), jnp.int32),
            pltpu.VMEM((GS, 2 * D), jnp.float32),
            pltpu.VMEM((GS, 2 * D), jnp.float32),
            pltpu.VMEM((GS // 2, 2 * D), jnp.float32),
            pltpu.SemaphoreType.DMA,
            pltpu.SemaphoreType.DMA,
        ],
    )
    def k(tok_hbm, tidx_hbm, aidx_hbm, peseg_hbm, out_hbm,
          tidx_v, aidx_v, tok_v, add_v, res_v, sem_t, sem_a):
        wid = lax.axis_index("s") * NC + lax.axis_index("c")
        pltpu.sync_copy(tidx_hbm.at[wid], tidx_v)
        pltpu.sync_copy(aidx_hbm.at[wid], aidx_v)

        def group(g, carry):
            gbase = g * GS
            cp_t = pltpu.async_copy(tok_hbm.at[tidx_v.at[pl.ds(gbase, GS)]],
                                    tok_v, sem_t)
            cp_a = pltpu.async_copy(peseg_hbm.at[aidx_v.at[pl.ds(gbase, GS)]],
                                    add_v, sem_a)
            cp_t.wait()
            cp_a.wait()

            def pair(rp, c2):
                for half in range(2):
                    r = 2 * rp + half
                    for c in range(D // LANES):
                        src = pl.ds(c * LANES, LANES)
                        dst = pl.ds(half * D + c * LANES, LANES)
                        res_v[rp, dst] = tok_v[r, src] + add_v[r, src]
                return c2

            lax.fori_loop(0, GS // 2, pair, 0, unroll=2)
            off = pl.multiple_of(wid * (TPW // 2) + g * (GS // 2), 8)
            pltpu.sync_copy(res_v, out_hbm.at[pl.ds(off, GS // 2)])
            return carry

        lax.fori_loop(0, NG, group, 0)

    return k(tok128, tidx2, aidx2, peseg)


def kernel(sequence, segment_labels, tok_table, seg_table, pe):
    tok128 = jnp.pad(tok_table, ((0, 0), (0, D)))
    tidx2 = sequence.astype(jnp.int32).reshape(NW, TPW)
    l_pos = jnp.arange(L, dtype=jnp.int32)
    aidx2 = (segment_labels.astype(jnp.int32) * L + l_pos[None, :]).reshape(NW, TPW)
    peseg = (seg_table[:, None, :] + pe[0, :L, :][None, :, :]).reshape(3 * L, D)
    peseg = jnp.concatenate([peseg, peseg], axis=1)
    out = _sc_embed(tok128, tidx2, aidx2, peseg)
    return out.reshape(B, L, D)
